# baseline (speedup 1.0000x reference)
import functools

import jax
import jax.numpy as jnp
from jax import lax
from jax.experimental import pallas as pl
from jax.experimental.pallas import tpu as pltpu
from jax.experimental.pallas import tpu_sc as plsc

_NUM_ROWS = 64
_ROW_STRIDE = 1024
_D = 512
_NUM_CORES = 2
_NUM_SUBCORES = 16
_NUM_WORKERS = _NUM_CORES * _NUM_SUBCORES
_ROWS_PER_WORKER = _NUM_ROWS // _NUM_WORKERS


@functools.partial(
    pl.kernel,
    mesh=plsc.VectorSubcoreMesh(core_axis_name="c", subcore_axis_name="s"),
    out_type=jax.ShapeDtypeStruct((_NUM_ROWS, _D), jnp.float32),
    scratch_types=[
        pltpu.SemaphoreType.DMA,
    ],
)
def _gather_rows(x_hbm, out_hbm, sem):
    wid = lax.axis_index("s") * _NUM_CORES + lax.axis_index("c")
    base = wid * _ROWS_PER_WORKER
    copies = []
    for j in range(_ROWS_PER_WORKER):
        copies.append(
            pltpu.make_async_copy(
                x_hbm.at[pl.ds((base + j) * _ROW_STRIDE, 1)],
                out_hbm.at[pl.ds(base + j, 1)],
                sem,
            )
        )
    for c in copies:
        c.start()
    for c in copies:
        c.wait()


def kernel(x):
    return _gather_rows(x)


# speedup vs baseline: 1.0809x; 1.0809x over previous
import functools

import jax
import jax.numpy as jnp
from jax import lax
from jax.experimental import pallas as pl
from jax.experimental.pallas import tpu as pltpu
from jax.experimental.pallas import tpu_sc as plsc

_NUM_ROWS = 64
_ROW_STRIDE = 1024
_D = 512
_NUM_CORES = 2
_ROWS_PER_CORE = _NUM_ROWS // _NUM_CORES


@functools.partial(
    pl.kernel,
    mesh=plsc.ScalarSubcoreMesh(axis_name="c", num_cores=_NUM_CORES),
    out_type=jax.ShapeDtypeStruct((_NUM_ROWS, _D), jnp.float32),
    scratch_types=[
        pltpu.SemaphoreType.DMA,
    ],
)
def _gather_rows(x_hbm, out_hbm, sem):
    cid = lax.axis_index("c")
    base = cid * _ROWS_PER_CORE
    copies = []
    for j in range(_ROWS_PER_CORE):
        copies.append(
            pltpu.make_async_copy(
                x_hbm.at[pl.ds((base + j) * _ROW_STRIDE, 1)],
                out_hbm.at[pl.ds(base + j, 1)],
                sem,
            )
        )
    for c in copies:
        c.start()
    for c in copies:
        c.wait()


def kernel(x):
    return _gather_rows(x)


# speedup vs baseline: 1.1435x; 1.0579x over previous
import functools

import jax
import jax.numpy as jnp
from jax import lax
from jax.experimental import pallas as pl
from jax.experimental.pallas import tpu as pltpu
from jax.experimental.pallas import tpu_sc as plsc

_NUM_ROWS = 64
_ROW_STRIDE = 1024
_D = 512
_NUM_CORES = 1
_ROWS_PER_CORE = _NUM_ROWS // _NUM_CORES


@functools.partial(
    pl.kernel,
    mesh=plsc.ScalarSubcoreMesh(axis_name="c", num_cores=_NUM_CORES),
    out_type=jax.ShapeDtypeStruct((_NUM_ROWS, _D), jnp.float32),
    scratch_types=[
        pltpu.SemaphoreType.DMA,
    ],
)
def _gather_rows(x_hbm, out_hbm, sem):
    cid = lax.axis_index("c")
    base = cid * _ROWS_PER_CORE
    copies = []
    for j in range(_ROWS_PER_CORE):
        copies.append(
            pltpu.make_async_copy(
                x_hbm.at[pl.ds((base + j) * _ROW_STRIDE, 1)],
                out_hbm.at[pl.ds(base + j, 1)],
                sem,
            )
        )
    for c in copies:
        c.start()
    for c in copies:
        c.wait()


def kernel(x):
    return _gather_rows(x)


# speedup vs baseline: 1.1711x; 1.0242x over previous
import functools

import jax
import jax.numpy as jnp
from jax import lax
from jax.experimental import pallas as pl
from jax.experimental.pallas import tpu as pltpu
from jax.experimental.pallas import tpu_sc as plsc

_NUM_ROWS = 64
_ROW_STRIDE = 1024
_D = 512
_NUM_CORES = 2
_NUM_SUBCORES = 16
_NUM_WORKERS = _NUM_CORES * _NUM_SUBCORES
_ROWS_PER_WORKER = _NUM_ROWS // _NUM_WORKERS


@functools.partial(
    pl.kernel,
    mesh=plsc.VectorSubcoreMesh(core_axis_name="c", subcore_axis_name="s"),
    out_type=jax.ShapeDtypeStruct((_NUM_ROWS, _D), jnp.float32),
    scratch_types=[
        pltpu.VMEM((_ROWS_PER_WORKER, _D), jnp.float32),
        pltpu.SemaphoreType.DMA,
    ],
)
def _gather_rows(x_hbm, out_hbm, buf, sem):
    wid = lax.axis_index("s") * _NUM_CORES + lax.axis_index("c")
    base = wid * _ROWS_PER_WORKER
    copies = [
        pltpu.make_async_copy(
            x_hbm.at[pl.ds((base + j) * _ROW_STRIDE, 1)],
            buf.at[pl.ds(j, 1)],
            sem,
        )
        for j in range(_ROWS_PER_WORKER)
    ]
    for c in copies:
        c.start()
    for c in copies:
        c.wait()
    pltpu.sync_copy(buf, out_hbm.at[pl.ds(base, _ROWS_PER_WORKER)])


def kernel(x):
    return _gather_rows(x)
